# baseline (speedup 1.0000x reference)
import functools

import jax
import jax.numpy as jnp
from jax import lax
from jax.experimental import pallas as pl
from jax.experimental.pallas import tpu as pltpu
from jax.experimental.pallas import tpu_sc as plsc

_S = 2048


def _routing_kernel(x_ref, wg_ref, meta_r_ref, meta_t_ref, xb_ref, *, S, E, C):
    x = x_ref[0]
    wg = wg_ref[...]
    l = jnp.dot(x, wg)
    m1 = jnp.max(l, axis=1, keepdims=True)
    p = jnp.exp(l - m1)
    p = p / jnp.sum(p, axis=1, keepdims=True)
    iota_e = lax.broadcasted_iota(jnp.int32, (S, E), 1).astype(jnp.float32)
    big = jnp.float32(E)
    pm1 = jnp.max(p, axis=1, keepdims=True)
    idx1 = jnp.min(jnp.where(p == pm1, iota_e, big), axis=1, keepdims=True)
    mask1 = (iota_e == idx1).astype(jnp.float32)
    g1 = jnp.sum(p * mask1, axis=1, keepdims=True)
    pwo = p * (1.0 - mask1)
    pm2 = jnp.max(pwo, axis=1, keepdims=True)
    idx2 = jnp.min(jnp.where(pwo == pm2, iota_e, big), axis=1, keepdims=True)
    mask2 = (iota_e == idx2).astype(jnp.float32)
    g2 = jnp.sum(pwo * mask2, axis=1, keepdims=True)

    ii = lax.broadcasted_iota(jnp.int32, (S, S), 0)
    jj = lax.broadcasted_iota(jnp.int32, (S, S), 1)
    tri = (jj < ii).astype(jnp.float32)
    cap = jnp.float32(C)
    trib = tri.astype(jnp.bfloat16)
    posm1 = jnp.dot(trib, mask1.astype(jnp.bfloat16),
                    preferred_element_type=jnp.float32)
    keep1 = mask1 * (posm1 < cap).astype(jnp.float32)
    pos1 = jnp.sum(posm1 * keep1, axis=1, keepdims=True)
    cnt1 = jnp.sum(keep1, axis=0, keepdims=True)
    flat1 = jnp.sum(keep1, axis=1, keepdims=True)
    posm2 = jnp.dot(trib, mask2.astype(jnp.bfloat16),
                    preferred_element_type=jnp.float32) + cnt1
    keep2 = mask2 * (posm2 < cap).astype(jnp.float32)
    pos2 = jnp.sum(posm2 * keep2, axis=1, keepdims=True)
    flat2 = jnp.sum(keep2, axis=1, keepdims=True)

    g1 = g1 * flat1
    g2 = g2 * flat2
    den = g1 + g2
    den = jnp.where(den > 0, den, jnp.float32(1.0))
    w1 = g1 / den * flat1
    w2 = g2 / den * flat2
    sent = jnp.float32(4 * S * E)
    slot1 = jnp.where(flat1 > 0, idx1 * cap + pos1, sent)
    slot2 = jnp.where(flat2 > 0, idx2 * cap + pos2, sent)

    cols = jnp.concatenate(
        [slot1, slot2, w1, w2, jnp.zeros((S, 4), jnp.float32)], axis=1)
    meta_t_ref[0] = cols
    zrow = jnp.zeros((1, S), jnp.float32)
    rows = jnp.concatenate(
        [jnp.reshape(slot1, (1, S)), jnp.reshape(slot2, (1, S)),
         jnp.reshape(w1, (1, S)), jnp.reshape(w2, (1, S)),
         zrow, zrow, zrow, zrow], axis=0)
    meta_r_ref[0] = rows
    xb_ref[0] = x.astype(jnp.bfloat16)


def _ffn_kernel(disp_ref, wi_ref, wo_ref, eo_ref, acc_ref,
                wib_ref, wob_ref, *, S, C, NHB):
    hb = pl.program_id(1)
    g = pl.program_id(2)

    @pl.when(g == 0)
    def _():
        wib_ref[...] = wi_ref[0].astype(jnp.bfloat16)
        wob_ref[...] = wo_ref[0].astype(jnp.bfloat16)

    h = jnp.maximum(jnp.dot(disp_ref[...], wib_ref[...],
                            preferred_element_type=jnp.float32), 0.0)
    contrib = jnp.dot(h.astype(jnp.bfloat16), wob_ref[...],
                      preferred_element_type=jnp.float32)

    @pl.when(hb == 0)
    def _():
        acc_ref[g] = contrib

    @pl.when(hb != 0)
    def _():
        acc_ref[g] += contrib

    @pl.when(hb == NHB - 1)
    def _():
        eo_ref[0, 0] = acc_ref[g].astype(jnp.bfloat16)


def _combine_kernel(meta_t_ref, eo_ref, out_ref, *, S, KB):
    kb = pl.program_id(1)
    slot1 = meta_t_ref[0, :, 0:1]
    slot2 = meta_t_ref[0, :, 1:2]
    w1 = meta_t_ref[0, :, 2:3]
    w2 = meta_t_ref[0, :, 3:4]
    iota_k = lax.broadcasted_iota(jnp.int32, (S, KB), 1).astype(jnp.float32)
    iota_k = iota_k + (kb * KB).astype(jnp.float32)
    wmat = (jnp.where(iota_k == slot1, w1, 0.0)
            + jnp.where(iota_k == slot2, w2, 0.0))
    contrib = jnp.dot(wmat.astype(jnp.bfloat16), eo_ref[0],
                      preferred_element_type=jnp.float32)

    @pl.when(kb == 0)
    def _():
        out_ref[0] = contrib

    @pl.when(kb != 0)
    def _():
        out_ref[0] += contrib


def _sc_dispatch(xb_hbm, meta_hbm, zrow_hbm, disp_hbm, rows_v, meta_v, idx_v,
                 sem, *, G, S, EC):
    cid = lax.axis_index("c")
    sid = lax.axis_index("s")
    wid = cid * 16 + sid
    tpw = (G * S) // 32
    rpw = (G * EC) // 32
    g = wid // (32 // G)
    tok_base = wid * tpw

    pltpu.sync_copy(zrow_hbm, rows_v)
    for ch in range(rpw // 128):
        pltpu.sync_copy(rows_v, disp_hbm.at[pl.ds(wid * rpw + ch * 128, 128)])
    plsc.subcore_barrier()

    big = jnp.float32(3 * S * 8)
    for ch in range(tpw // 128):
        off = tok_base + ch * 128
        pltpu.sync_copy(xb_hbm.at[pl.ds(off, 128)], rows_v)
        soff = off - g * S
        pltpu.sync_copy(meta_hbm.at[g, pl.ds(0, 2), pl.ds(soff, 128)], meta_v)
        for r in range(2):
            for k in range(8):
                sl = meta_v[r, pl.ds(k * 16, 16)]
                gi = sl.astype(jnp.int32) + g * EC
                gi = jnp.where(sl < big, gi, jnp.int32(G * EC))
                idx_v[pl.ds(k * 16, 16)] = gi
            pltpu.async_copy(rows_v, disp_hbm.at[idx_v], sem).wait()


def kernel(inputs, wg, wi, wo):
    n, M = inputs.shape
    S = _S
    G = n // S
    E = wi.shape[0]
    H = wi.shape[2]
    C = 2 * S // E
    HB = H // 4
    KB = 1024
    x = inputs.reshape(G, S, M)

    meta_r, meta_t, xb = pl.pallas_call(
        functools.partial(_routing_kernel, S=S, E=E, C=C),
        grid=(G,),
        in_specs=[
            pl.BlockSpec((1, S, M), lambda g: (g, 0, 0)),
            pl.BlockSpec((M, E), lambda g: (0, 0)),
        ],
        out_specs=[
            pl.BlockSpec((1, 8, S), lambda g: (g, 0, 0)),
            pl.BlockSpec((1, S, 8), lambda g: (g, 0, 0)),
            pl.BlockSpec((1, S, M), lambda g: (g, 0, 0)),
        ],
        out_shape=[
            jax.ShapeDtypeStruct((G, 8, S), jnp.float32),
            jax.ShapeDtypeStruct((G, S, 8), jnp.float32),
            jax.ShapeDtypeStruct((G, S, M), jnp.bfloat16),
        ],
    )(x, wg)

    EC = E * C
    mesh = plsc.VectorSubcoreMesh(core_axis_name="c", subcore_axis_name="s")
    xb32 = lax.bitcast_convert_type(
        xb.reshape(G * S, M // 2, 2), jnp.int32)
    zrow = jnp.zeros((128, M // 2), jnp.int32)
    disp32 = pl.kernel(
        functools.partial(_sc_dispatch, G=G, S=S, EC=EC),
        mesh=mesh,
        out_type=jax.ShapeDtypeStruct((G * EC + C, M // 2), jnp.int32),
        scratch_types=[pltpu.VMEM((128, M // 2), jnp.int32),
                       pltpu.VMEM((2, 128), jnp.float32),
                       pltpu.VMEM((128,), jnp.int32),
                       pltpu.SemaphoreType.DMA],
    )(xb32, meta_r, zrow)
    disp = lax.bitcast_convert_type(disp32, jnp.bfloat16)
    disp = disp.reshape(G * EC + C, M)

    eo = pl.pallas_call(
        functools.partial(_ffn_kernel, S=S, C=C, NHB=H // HB),
        grid=(E, H // HB, G),
        in_specs=[
            pl.BlockSpec((C, M), lambda e, hb, g: (g * E + e, 0)),
            pl.BlockSpec((1, M, HB), lambda e, hb, g: (e, 0, hb)),
            pl.BlockSpec((1, HB, M), lambda e, hb, g: (e, hb, 0)),
        ],
        out_specs=pl.BlockSpec((1, 1, C, M), lambda e, hb, g: (g, e, 0, 0)),
        out_shape=jax.ShapeDtypeStruct((G, E, C, M), jnp.bfloat16),
        scratch_shapes=[pltpu.VMEM((G, C, M), jnp.float32),
                        pltpu.VMEM((M, HB), jnp.bfloat16),
                        pltpu.VMEM((HB, M), jnp.bfloat16)],
    )(disp, wi, wo)

    eo = eo.reshape(G, E * C, M)
    out = pl.pallas_call(
        functools.partial(_combine_kernel, S=S, KB=KB),
        grid=(G, (E * C) // KB),
        in_specs=[
            pl.BlockSpec((1, S, 8), lambda g, kb: (g, 0, 0)),
            pl.BlockSpec((1, KB, M), lambda g, kb: (g, kb, 0)),
        ],
        out_specs=pl.BlockSpec((1, S, M), lambda g, kb: (g, 0, 0)),
        out_shape=jax.ShapeDtypeStruct((G, S, M), jnp.float32),
    )(meta_t, eo)
    return out.reshape(n, M)


# speedup vs baseline: 1.8629x; 1.8629x over previous
import functools

import jax
import jax.numpy as jnp
from jax import lax
from jax.experimental import pallas as pl
from jax.experimental.pallas import tpu as pltpu

_S = 2048


def _routing_kernel(x_ref, wg_ref, meta_r_ref, meta_t_ref, xb_ref, *, S, E, C):
    x = x_ref[0]
    wg = wg_ref[...]
    l = jnp.dot(x, wg)
    m1 = jnp.max(l, axis=1, keepdims=True)
    p = jnp.exp(l - m1)
    p = p / jnp.sum(p, axis=1, keepdims=True)
    iota_e = lax.broadcasted_iota(jnp.int32, (S, E), 1).astype(jnp.float32)
    big = jnp.float32(E)
    pm1 = jnp.max(p, axis=1, keepdims=True)
    idx1 = jnp.min(jnp.where(p == pm1, iota_e, big), axis=1, keepdims=True)
    mask1 = (iota_e == idx1).astype(jnp.float32)
    g1 = jnp.sum(p * mask1, axis=1, keepdims=True)
    pwo = p * (1.0 - mask1)
    pm2 = jnp.max(pwo, axis=1, keepdims=True)
    idx2 = jnp.min(jnp.where(pwo == pm2, iota_e, big), axis=1, keepdims=True)
    mask2 = (iota_e == idx2).astype(jnp.float32)
    g2 = jnp.sum(pwo * mask2, axis=1, keepdims=True)

    ii = lax.broadcasted_iota(jnp.int32, (S, S), 0)
    jj = lax.broadcasted_iota(jnp.int32, (S, S), 1)
    tri = (jj < ii).astype(jnp.float32)
    cap = jnp.float32(C)
    trib = tri.astype(jnp.bfloat16)
    posm1 = jnp.dot(trib, mask1.astype(jnp.bfloat16),
                    preferred_element_type=jnp.float32)
    keep1 = mask1 * (posm1 < cap).astype(jnp.float32)
    pos1 = jnp.sum(posm1 * keep1, axis=1, keepdims=True)
    cnt1 = jnp.sum(keep1, axis=0, keepdims=True)
    flat1 = jnp.sum(keep1, axis=1, keepdims=True)
    posm2 = jnp.dot(trib, mask2.astype(jnp.bfloat16),
                    preferred_element_type=jnp.float32) + cnt1
    keep2 = mask2 * (posm2 < cap).astype(jnp.float32)
    pos2 = jnp.sum(posm2 * keep2, axis=1, keepdims=True)
    flat2 = jnp.sum(keep2, axis=1, keepdims=True)

    g1 = g1 * flat1
    g2 = g2 * flat2
    den = g1 + g2
    den = jnp.where(den > 0, den, jnp.float32(1.0))
    w1 = g1 / den * flat1
    w2 = g2 / den * flat2
    sent = jnp.float32(4 * S * E)
    slot1 = jnp.where(flat1 > 0, idx1 * cap + pos1, sent)
    slot2 = jnp.where(flat2 > 0, idx2 * cap + pos2, sent)

    cols = jnp.concatenate(
        [slot1, slot2, w1, w2, jnp.zeros((S, 4), jnp.float32)], axis=1)
    meta_t_ref[0] = cols
    zrow = jnp.zeros((1, S), jnp.float32)
    rows = jnp.concatenate(
        [jnp.reshape(slot1, (1, S)), jnp.reshape(slot2, (1, S)),
         jnp.reshape(w1, (1, S)), jnp.reshape(w2, (1, S)),
         zrow, zrow, zrow, zrow], axis=0)
    meta_r_ref[0] = rows
    xb_ref[0] = x.astype(jnp.bfloat16)


def _ffn_kernel(meta_r_ref, x_ref, wi_ref, wo_ref, eo_ref, disp_ref, acc_ref,
                wib_ref, wob_ref, *, S, C, NHB):
    e = pl.program_id(0)
    hb = pl.program_id(1)
    g = pl.program_id(2)

    @pl.when(g == 0)
    def _():
        wib_ref[...] = wi_ref[0].astype(jnp.bfloat16)
        wob_ref[...] = wo_ref[0].astype(jnp.bfloat16)

    @pl.when(hb == 0)
    def _():
        slot1 = meta_r_ref[g, 0:1, :]
        slot2 = meta_r_ref[g, 1:2, :]
        iota_c = lax.broadcasted_iota(jnp.int32, (C, S), 0).astype(jnp.float32)
        iota_c = iota_c + (e * C).astype(jnp.float32)
        oh = jnp.logical_or(iota_c == slot1, iota_c == slot2)
        disp_ref[g] = jnp.dot(oh.astype(jnp.bfloat16), x_ref[g],
                              preferred_element_type=jnp.float32
                              ).astype(jnp.bfloat16)

    h = jnp.maximum(jnp.dot(disp_ref[g], wib_ref[...],
                            preferred_element_type=jnp.float32), 0.0)
    contrib = jnp.dot(h.astype(jnp.bfloat16), wob_ref[...],
                      preferred_element_type=jnp.float32)

    @pl.when(hb == 0)
    def _():
        acc_ref[g] = contrib

    @pl.when(hb != 0)
    def _():
        acc_ref[g] += contrib

    @pl.when(hb == NHB - 1)
    def _():
        eo_ref[0, 0] = acc_ref[g].astype(jnp.bfloat16)


def _combine_kernel(meta_t_ref, eo_ref, out_ref, *, S, KB):
    kb = pl.program_id(1)
    slot1 = meta_t_ref[0, :, 0:1]
    slot2 = meta_t_ref[0, :, 1:2]
    w1 = meta_t_ref[0, :, 2:3]
    w2 = meta_t_ref[0, :, 3:4]
    iota_k = lax.broadcasted_iota(jnp.int32, (S, KB), 1).astype(jnp.float32)
    iota_k = iota_k + (kb * KB).astype(jnp.float32)
    wmat = (jnp.where(iota_k == slot1, w1, 0.0)
            + jnp.where(iota_k == slot2, w2, 0.0))
    contrib = jnp.dot(wmat.astype(jnp.bfloat16), eo_ref[0],
                      preferred_element_type=jnp.float32)

    @pl.when(kb == 0)
    def _():
        out_ref[0] = contrib

    @pl.when(kb != 0)
    def _():
        out_ref[0] += contrib


def kernel(inputs, wg, wi, wo):
    n, M = inputs.shape
    S = _S
    G = n // S
    E = wi.shape[0]
    H = wi.shape[2]
    C = 2 * S // E
    HB = H // 4
    KB = 1024
    x = inputs.reshape(G, S, M)

    meta_r, meta_t, xb = pl.pallas_call(
        functools.partial(_routing_kernel, S=S, E=E, C=C),
        grid=(G,),
        in_specs=[
            pl.BlockSpec((1, S, M), lambda g: (g, 0, 0)),
            pl.BlockSpec((M, E), lambda g: (0, 0)),
        ],
        out_specs=[
            pl.BlockSpec((1, 8, S), lambda g: (g, 0, 0)),
            pl.BlockSpec((1, S, 8), lambda g: (g, 0, 0)),
            pl.BlockSpec((1, S, M), lambda g: (g, 0, 0)),
        ],
        out_shape=[
            jax.ShapeDtypeStruct((G, 8, S), jnp.float32),
            jax.ShapeDtypeStruct((G, S, 8), jnp.float32),
            jax.ShapeDtypeStruct((G, S, M), jnp.bfloat16),
        ],
    )(x, wg)

    eo = pl.pallas_call(
        functools.partial(_ffn_kernel, S=S, C=C, NHB=H // HB),
        grid=(E, H // HB, G),
        in_specs=[
            pl.BlockSpec((G, 8, S), lambda e, hb, g: (0, 0, 0)),
            pl.BlockSpec((G, S, M), lambda e, hb, g: (0, 0, 0)),
            pl.BlockSpec((1, M, HB), lambda e, hb, g: (e, 0, hb)),
            pl.BlockSpec((1, HB, M), lambda e, hb, g: (e, hb, 0)),
        ],
        out_specs=pl.BlockSpec((1, 1, C, M), lambda e, hb, g: (g, e, 0, 0)),
        out_shape=jax.ShapeDtypeStruct((G, E, C, M), jnp.bfloat16),
        scratch_shapes=[pltpu.VMEM((G, C, M), jnp.bfloat16),
                        pltpu.VMEM((G, C, M), jnp.float32),
                        pltpu.VMEM((M, HB), jnp.bfloat16),
                        pltpu.VMEM((HB, M), jnp.bfloat16)],
    )(meta_r, xb, wi, wo)

    eo = eo.reshape(G, E * C, M)
    out = pl.pallas_call(
        functools.partial(_combine_kernel, S=S, KB=KB),
        grid=(G, (E * C) // KB),
        in_specs=[
            pl.BlockSpec((1, S, 8), lambda g, kb: (g, 0, 0)),
            pl.BlockSpec((1, KB, M), lambda g, kb: (g, kb, 0)),
        ],
        out_specs=pl.BlockSpec((1, S, M), lambda g, kb: (g, 0, 0)),
        out_shape=jax.ShapeDtypeStruct((G, S, M), jnp.float32),
    )(meta_t, eo)
    return out.reshape(n, M)
